# Initial kernel scaffold; baseline (speedup 1.0000x reference)
import jax
import jax.numpy as jnp
from jax.experimental import pallas as pl


def kernel(x, row_splits, W1_s, b1_s, W1_h, b1_h, W1_o, b1_o, W2_s, b2_s, W2_h, b2_h, W2_o, b2_o, Wb1, bb1, Wb2, bb2, Wb3, bb3):
    raise NotImplementedError("write your pallas kernel here")


# speedup vs baseline: 1.2202x; 1.2202x over previous
import functools

import jax
import jax.numpy as jnp
from jax.experimental import pallas as pl
from jax.experimental.pallas import tpu as pltpu

_K = 16
_PEN = 1e6


def _proj_body(x_ref, rs_ref, Ws_ref, bs_ref, Wh_ref, bh_ref, sb_ref, g_ref):
    x = x_ref[...]
    s = jnp.dot(x, Ws_ref[...], preferred_element_type=jnp.float32) + bs_ref[...]
    h = jnp.dot(x, Wh_ref[...], preferred_element_type=jnp.float32) + bh_ref[...]
    sq = jnp.sum(s * s, axis=1, keepdims=True)
    n = x.shape[0]
    r_iota = jax.lax.broadcasted_iota(jnp.int32, (n, 1), 0)
    batch = jnp.zeros((n, 1), jnp.int32)
    for t in range(rs_ref.shape[0]):
        batch = batch + (rs_ref[t] <= r_iota).astype(jnp.int32)
    sb_ref[...] = jnp.concatenate([s, sq], axis=1)
    g_ref[...] = jnp.concatenate([s, h, batch.astype(jnp.float32)], axis=1)


def _proj(x, rs, Ws, bs, Wh, bh):
    n = x.shape[0]
    s_dim = Ws.shape[1]
    f_dim = Wh.shape[1]
    return pl.pallas_call(
        _proj_body,
        in_specs=[
            pl.BlockSpec(memory_space=pltpu.VMEM),
            pl.BlockSpec(memory_space=pltpu.SMEM),
            pl.BlockSpec(memory_space=pltpu.VMEM),
            pl.BlockSpec(memory_space=pltpu.VMEM),
            pl.BlockSpec(memory_space=pltpu.VMEM),
            pl.BlockSpec(memory_space=pltpu.VMEM),
        ],
        out_shape=(
            jax.ShapeDtypeStruct((n, s_dim + 1), jnp.float32),
            jax.ShapeDtypeStruct((n, s_dim + f_dim + 1), jnp.float32),
        ),
    )(x, rs, Ws, bs.reshape(1, -1), Wh, bh.reshape(1, -1))


def _grav_body(bm, head, *refs):
    if head:
        (x_ref, sb_ref, g_ref, rs_ref, Wo_ref, bo_ref,
         Wb1_ref, bb1_ref, Wb2_ref, bb2_ref, Wb3_ref, bb3_ref,
         out_ref, beta_ref) = refs
    else:
        (x_ref, sb_ref, g_ref, rs_ref, Wo_ref, bo_ref, out_ref) = refs

    i = pl.program_id(0)
    x = x_ref[...]
    sb = sb_ref[...]
    g = g_ref[...]
    n = sb.shape[0]
    s_dim = sb.shape[1] - 1
    f_dim = g.shape[1] - s_dim - 1
    d_dim = x.shape[1]

    sb_blk = sb_ref[pl.ds(i * bm, bm), :]
    s_blk = sb_blk[:, :s_dim]
    sq_blk = sb_blk[:, s_dim:]

    c_iota = jax.lax.broadcasted_iota(jnp.int32, (1, n), 1)
    r_iota = jax.lax.broadcasted_iota(jnp.int32, (bm, 1), 0) + i * bm
    batch_r = jnp.zeros((bm, 1), jnp.int32)
    batch_c = jnp.zeros((1, n), jnp.int32)
    for t in range(rs_ref.shape[0]):
        rv = rs_ref[t]
        batch_r = batch_r + (rv <= r_iota).astype(jnp.int32)
        batch_c = batch_c + (rv <= c_iota).astype(jnp.int32)
    pen = jnp.where(batch_r != batch_c, jnp.float32(_PEN), jnp.float32(0.0))
    batch_rf = batch_r.astype(jnp.float32)

    a_blk = jnp.concatenate(
        [-2.0 * s_blk, jnp.ones((bm, 1), jnp.float32)], axis=1)
    d2 = jax.lax.dot_general(
        a_blk, sb, (((1,), (1,)), ((), ())),
        preferred_element_type=jnp.float32) + sq_blk
    d2p = d2 + pen

    inf = jnp.float32(jnp.inf)

    def body(_, carry):
        d2p, acc_sum, acc_max = carry
        minval = jnp.min(d2p, axis=1, keepdims=True)
        eq = d2p == minval
        sel = jnp.min(jnp.where(eq, c_iota, n), axis=1, keepdims=True)
        onehot = c_iota == sel
        g_sel = jnp.dot(onehot.astype(jnp.float32), g,
                        preferred_element_type=jnp.float32,
                        precision=jax.lax.Precision.HIGHEST)
        s_nb = g_sel[:, :s_dim]
        h_sel = g_sel[:, s_dim:s_dim + f_dim]
        b_sel = g_sel[:, s_dim + f_dim:]
        diff = s_blk - s_nb
        d2_sel = jnp.sum(diff * diff, axis=1, keepdims=True)
        pen_sel = jnp.where(batch_rf != b_sel, jnp.float32(_PEN),
                            jnp.float32(0.0))
        w = jnp.exp(-10.0 * (d2_sel + pen_sel))
        msg = h_sel * w
        acc_sum = acc_sum + msg
        acc_max = jnp.maximum(acc_max, msg)
        d2p = jnp.where(onehot, inf, d2p)
        return d2p, acc_sum, acc_max

    _, acc_sum, acc_max = jax.lax.fori_loop(
        0, _K, body,
        (d2p,
         jnp.zeros((bm, f_dim), jnp.float32),
         jnp.full((bm, f_dim), -jnp.inf, jnp.float32)))
    mean = acc_sum * (1.0 / _K)

    Wo = Wo_ref[...]
    out = (jnp.dot(x, Wo[:d_dim], preferred_element_type=jnp.float32)
           + jnp.dot(mean, Wo[d_dim:d_dim + f_dim],
                     preferred_element_type=jnp.float32)
           + jnp.dot(acc_max, Wo[d_dim + f_dim:],
                     preferred_element_type=jnp.float32)
           + bo_ref[...])
    out_ref[...] = out

    if head:
        hb = jnp.maximum(
            jnp.dot(out, Wb1_ref[...], preferred_element_type=jnp.float32)
            + bb1_ref[...], 0.0)
        hb = jnp.maximum(
            jnp.dot(hb, Wb2_ref[...], preferred_element_type=jnp.float32)
            + bb2_ref[...], 0.0)
        logit = (jnp.dot(hb, Wb3_ref[...], preferred_element_type=jnp.float32)
                 + bb3_ref[...])
        beta = jax.nn.sigmoid(logit)
        beta_ref[...] = jnp.clip(beta, 1e-6, 1.0 - 1e-6)


def _pick_bm(n):
    for bm in (80, 64, 40, 16, 8):
        if n % bm == 0 and bm % 8 == 0:
            return bm
    return n


def _grav_layer(x, sb, g, rs, Wo, bo, head_params=None):
    n, d_dim = x.shape
    s1 = sb.shape[1]
    g1 = g.shape[1]
    bm = _pick_bm(n)
    grid = (n // bm,)

    full = lambda shape: pl.BlockSpec(shape, lambda i: (0, 0))
    in_specs = [
        pl.BlockSpec((bm, d_dim), lambda i: (i, 0)),
        full((n, s1)),
        full((n, g1)),
        pl.BlockSpec(memory_space=pltpu.SMEM),
        full(Wo.shape),
        full((1, d_dim)),
    ]
    args = [x, sb, g, rs, Wo, bo.reshape(1, -1)]
    if head_params is None:
        out_shape = jax.ShapeDtypeStruct((n, d_dim), jnp.float32)
        out_specs = pl.BlockSpec((bm, d_dim), lambda i: (i, 0))
    else:
        Wb1, bb1, Wb2, bb2, Wb3, bb3 = head_params
        in_specs += [full(Wb1.shape), full((1, Wb1.shape[1])),
                     full(Wb2.shape), full((1, Wb2.shape[1])),
                     full(Wb3.shape), full((1, Wb3.shape[1]))]
        args += [Wb1, bb1.reshape(1, -1), Wb2, bb2.reshape(1, -1),
                 Wb3, bb3.reshape(1, -1)]
        out_shape = (jax.ShapeDtypeStruct((n, d_dim), jnp.float32),
                     jax.ShapeDtypeStruct((n, 1), jnp.float32))
        out_specs = (pl.BlockSpec((bm, d_dim), lambda i: (i, 0)),
                     pl.BlockSpec((bm, 1), lambda i: (i, 0)))

    return pl.pallas_call(
        functools.partial(_grav_body, bm, head_params is not None),
        grid=grid,
        in_specs=in_specs,
        out_specs=out_specs,
        out_shape=out_shape,
    )(*args)


def kernel(x, row_splits, W1_s, b1_s, W1_h, b1_h, W1_o, b1_o,
           W2_s, b2_s, W2_h, b2_h, W2_o, b2_o,
           Wb1, bb1, Wb2, bb2, Wb3, bb3):
    rs = row_splits.astype(jnp.int32)
    sb1, g1 = _proj(x, rs, W1_s, b1_s, W1_h, b1_h)
    lat1 = _grav_layer(x, sb1, g1, rs, W1_o, b1_o)
    sb2, g2 = _proj(lat1, rs, W2_s, b2_s, W2_h, b2_h)
    latent, beta = _grav_layer(lat1, sb2, g2, rs, W2_o, b2_o,
                               head_params=(Wb1, bb1, Wb2, bb2, Wb3, bb3))
    return beta, latent


# speedup vs baseline: 2.5333x; 2.0762x over previous
import functools

import jax
import jax.numpy as jnp
from jax.experimental import pallas as pl
from jax.experimental.pallas import tpu as pltpu

_K = 16
_PEN = 1e6


def _proj_body(x_ref, Ws_ref, bs_ref, Wh_ref, bh_ref, sb_ref, h_ref):
    x = x_ref[...]
    s = jnp.dot(x, Ws_ref[...], preferred_element_type=jnp.float32,
                precision=jax.lax.Precision.HIGHEST) + bs_ref[...]
    h = jnp.dot(x, Wh_ref[...], preferred_element_type=jnp.float32,
                precision=jax.lax.Precision.HIGHEST) + bh_ref[...]
    sq = jnp.sum(s * s, axis=1, keepdims=True)
    sb_ref[...] = jnp.concatenate([s, sq], axis=1)
    h_ref[...] = h


def _proj(x, Ws, bs, Wh, bh):
    n = x.shape[0]
    s_dim = Ws.shape[1]
    f_dim = Wh.shape[1]
    return pl.pallas_call(
        _proj_body,
        out_shape=(
            jax.ShapeDtypeStruct((n, s_dim + 1), jnp.float32),
            jax.ShapeDtypeStruct((n, f_dim), jnp.float32),
        ),
    )(x, Ws, bs.reshape(1, -1), Wh, bh.reshape(1, -1))


def _grav_body(bm, head, *refs):
    if head:
        (x_ref, sb_ref, h_ref, rs_ref, Wo_ref, bo_ref,
         Wb1_ref, bb1_ref, Wb2_ref, bb2_ref, Wb3_ref, bb3_ref,
         out_ref, beta_ref) = refs
    else:
        (x_ref, sb_ref, h_ref, rs_ref, Wo_ref, bo_ref, out_ref) = refs

    i = pl.program_id(0)
    x = x_ref[...]
    sb = sb_ref[...]
    h_all = h_ref[...]
    n = sb.shape[0]
    s_dim = sb.shape[1] - 1
    f_dim = h_all.shape[1]
    d_dim = x.shape[1]

    sb_blk = sb_ref[pl.ds(i * bm, bm), :]
    s_blk = sb_blk[:, :s_dim]
    sq_blk = sb_blk[:, s_dim:]

    c_iota = jax.lax.broadcasted_iota(jnp.int32, (1, n), 1)
    r_iota = jax.lax.broadcasted_iota(jnp.int32, (bm, 1), 0) + i * bm
    batch_r = jnp.zeros((bm, 1), jnp.int32)
    batch_c = jnp.zeros((1, n), jnp.int32)
    for t in range(rs_ref.shape[0]):
        rv = rs_ref[t]
        batch_r = batch_r + (rv <= r_iota).astype(jnp.int32)
        batch_c = batch_c + (rv <= c_iota).astype(jnp.int32)
    pen = jnp.where(batch_r != batch_c, jnp.float32(_PEN), jnp.float32(0.0))

    a_blk = jnp.concatenate(
        [-2.0 * s_blk, jnp.ones((bm, 1), jnp.float32)], axis=1)
    d2 = jax.lax.dot_general(
        a_blk, sb, (((1,), (1,)), ((), ())),
        preferred_element_type=jnp.float32,
        precision=jax.lax.Precision.HIGHEST) + sq_blk
    d2p = d2 + pen

    inf = jnp.float32(jnp.inf)

    def body(_, carry):
        d2p, acc_sum, acc_max = carry
        minval = jnp.min(d2p, axis=1, keepdims=True)
        eq = d2p == minval
        sel = jnp.min(jnp.where(eq, c_iota, n), axis=1, keepdims=True)
        onehot = c_iota == sel
        h_sel = jnp.dot(onehot.astype(jnp.float32), h_all,
                        preferred_element_type=jnp.float32)
        w = jnp.exp(-10.0 * minval)
        msg = h_sel * w
        acc_sum = acc_sum + msg
        acc_max = jnp.maximum(acc_max, msg)
        d2p = jnp.where(onehot, inf, d2p)
        return d2p, acc_sum, acc_max

    _, acc_sum, acc_max = jax.lax.fori_loop(
        0, _K, body,
        (d2p,
         jnp.zeros((bm, f_dim), jnp.float32),
         jnp.full((bm, f_dim), -jnp.inf, jnp.float32)))
    mean = acc_sum * (1.0 / _K)

    Wo = Wo_ref[...]
    out = (jnp.dot(x, Wo[:d_dim], preferred_element_type=jnp.float32)
           + jnp.dot(mean, Wo[d_dim:d_dim + f_dim],
                     preferred_element_type=jnp.float32)
           + jnp.dot(acc_max, Wo[d_dim + f_dim:],
                     preferred_element_type=jnp.float32)
           + bo_ref[...])
    out_ref[...] = out

    if head:
        hb = jnp.maximum(
            jnp.dot(out, Wb1_ref[...], preferred_element_type=jnp.float32)
            + bb1_ref[...], 0.0)
        hb = jnp.maximum(
            jnp.dot(hb, Wb2_ref[...], preferred_element_type=jnp.float32)
            + bb2_ref[...], 0.0)
        logit = (jnp.dot(hb, Wb3_ref[...], preferred_element_type=jnp.float32)
                 + bb3_ref[...])
        beta = jax.nn.sigmoid(logit)
        beta_ref[...] = jnp.clip(beta, 1e-6, 1.0 - 1e-6)


def _pick_bm(n):
    for bm in (80, 64, 40, 16, 8):
        if n % bm == 0 and bm % 8 == 0:
            return bm
    return n


def _grav_layer(x, sb, h, rs, Wo, bo, head_params=None):
    n, d_dim = x.shape
    s1 = sb.shape[1]
    f_dim = h.shape[1]
    bm = _pick_bm(n)
    grid = (n // bm,)

    full = lambda shape: pl.BlockSpec(shape, lambda i: (0, 0))
    in_specs = [
        pl.BlockSpec((bm, d_dim), lambda i: (i, 0)),
        full((n, s1)),
        full((n, f_dim)),
        pl.BlockSpec(memory_space=pltpu.SMEM),
        full(Wo.shape),
        full((1, d_dim)),
    ]
    args = [x, sb, h, rs, Wo, bo.reshape(1, -1)]
    if head_params is None:
        out_shape = jax.ShapeDtypeStruct((n, d_dim), jnp.float32)
        out_specs = pl.BlockSpec((bm, d_dim), lambda i: (i, 0))
    else:
        Wb1, bb1, Wb2, bb2, Wb3, bb3 = head_params
        in_specs += [full(Wb1.shape), full((1, Wb1.shape[1])),
                     full(Wb2.shape), full((1, Wb2.shape[1])),
                     full(Wb3.shape), full((1, Wb3.shape[1]))]
        args += [Wb1, bb1.reshape(1, -1), Wb2, bb2.reshape(1, -1),
                 Wb3, bb3.reshape(1, -1)]
        out_shape = (jax.ShapeDtypeStruct((n, d_dim), jnp.float32),
                     jax.ShapeDtypeStruct((n, 1), jnp.float32))
        out_specs = (pl.BlockSpec((bm, d_dim), lambda i: (i, 0)),
                     pl.BlockSpec((bm, 1), lambda i: (i, 0)))

    return pl.pallas_call(
        functools.partial(_grav_body, bm, head_params is not None),
        grid=grid,
        in_specs=in_specs,
        out_specs=out_specs,
        out_shape=out_shape,
    )(*args)


def kernel(x, row_splits, W1_s, b1_s, W1_h, b1_h, W1_o, b1_o,
           W2_s, b2_s, W2_h, b2_h, W2_o, b2_o,
           Wb1, bb1, Wb2, bb2, Wb3, bb3):
    rs = row_splits.astype(jnp.int32)
    sb1, h1 = _proj(x, W1_s, b1_s, W1_h, b1_h)
    lat1 = _grav_layer(x, sb1, h1, rs, W1_o, b1_o)
    sb2, h2 = _proj(lat1, W2_s, b2_s, W2_h, b2_h)
    latent, beta = _grav_layer(lat1, sb2, h2, rs, W2_o, b2_o,
                               head_params=(Wb1, bb1, Wb2, bb2, Wb3, bb3))
    return beta, latent
